# Initial kernel scaffold; baseline (speedup 1.0000x reference)
import jax
import jax.numpy as jnp
from jax.experimental import pallas as pl


def kernel(cards, discard_pile, hand_fc1_w, hand_fc1_b, hand_fc2_w, hand_fc2_b, gru_w_ih, gru_w_hh, gru_b_ih, gru_b_hh, dp_fc1_w, dp_fc1_b):
    raise NotImplementedError("write your pallas kernel here")


# speedup vs baseline: 1.9220x; 1.9220x over previous
import jax
import jax.numpy as jnp
from jax.experimental import pallas as pl
from jax.experimental.pallas import tpu as pltpu

_B = 256
_T = 52
_IN = 52
_NA = 13 * 4 * 13 * 2
_H = _NA * 2
_HAND = 13 * 4 * 13

_F32 = jnp.float32
_BF16 = jnp.bfloat16


def _gru_body(seq_ref,
              wih_r_ref, wih_z_ref, wih_n_ref,
              whh_r_ref, whh_z_ref, whh_n_ref,
              br_ref, bz_ref, bin_ref, bhn_ref,
              h_ref, mask_ref, valid_scr):
    h_ref[...] = jnp.zeros((_B, _H), _F32)
    valid_scr[...] = jnp.ones((_B, 1), _F32)

    def step(t, carry):
        x = seq_ref[t]
        nz = jnp.sum(x.astype(_F32), axis=1, keepdims=True) != 0.0
        v = jnp.logical_and(valid_scr[...] > 0.0, nz)
        valid_scr[...] = v.astype(_F32)

        @pl.when(t == 0)
        def _():
            mask_ref[...] = nz.astype(jnp.int32)

        h = h_ref[...]
        hb = h.astype(_BF16)
        r = jax.nn.sigmoid(
            jnp.dot(x, wih_r_ref[...], preferred_element_type=_F32)
            + jnp.dot(hb, whh_r_ref[...], preferred_element_type=_F32)
            + br_ref[...])
        n = jnp.tanh(
            jnp.dot(x, wih_n_ref[...], preferred_element_type=_F32)
            + bin_ref[...]
            + r * (jnp.dot(hb, whh_n_ref[...], preferred_element_type=_F32)
                   + bhn_ref[...]))
        z = jax.nn.sigmoid(
            jnp.dot(x, wih_z_ref[...], preferred_element_type=_F32)
            + jnp.dot(hb, whh_z_ref[...], preferred_element_type=_F32)
            + bz_ref[...])
        h_ref[...] = jnp.where(v, n + z * (h - n), h)
        return carry

    jax.lax.fori_loop(0, _T, step, 0)


def _head_body(cards_ref, h_ref, len_ref,
               w1_ref, b1_ref, w2_ref, b2_ref, wdp_ref, bdp_ref,
               y_ref):
    xh = jnp.maximum(
        jnp.dot(cards_ref[...], w1_ref[...], preferred_element_type=_F32)
        + b1_ref[...], 0.0)
    yh = jnp.dot(xh.astype(_BF16), w2_ref[...], preferred_element_type=_F32) \
        + b2_ref[...]
    xdp = jnp.dot(h_ref[...].astype(_BF16), wdp_ref[...],
                  preferred_element_type=_F32) + bdp_ref[...]
    mask = len_ref[...] > 0
    y_ref[...] = jnp.where(mask, 0.3 * yh + 0.7 * xdp, yh)


def kernel(cards, discard_pile, hand_fc1_w, hand_fc1_b, hand_fc2_w, hand_fc2_b,
           gru_w_ih, gru_w_hh, gru_b_ih, gru_b_hh, dp_fc1_w, dp_fc1_b):
    seq_t = jnp.transpose(discard_pile.reshape(_B, _T, _IN),
                          (1, 0, 2)).astype(_BF16)

    wih_r = gru_w_ih[:_H].T.astype(_BF16)
    wih_z = gru_w_ih[_H:2 * _H].T.astype(_BF16)
    wih_n = gru_w_ih[2 * _H:].T.astype(_BF16)
    whh_r = gru_w_hh[:_H].T.astype(_BF16)
    whh_z = gru_w_hh[_H:2 * _H].T.astype(_BF16)
    whh_n = gru_w_hh[2 * _H:].T.astype(_BF16)
    br = (gru_b_ih[:_H] + gru_b_hh[:_H]).reshape(1, _H)
    bz = (gru_b_ih[_H:2 * _H] + gru_b_hh[_H:2 * _H]).reshape(1, _H)
    b_in = gru_b_ih[2 * _H:].reshape(1, _H)
    b_hn = gru_b_hh[2 * _H:].reshape(1, _H)

    h_last, mask = pl.pallas_call(
        _gru_body,
        out_shape=[
            jax.ShapeDtypeStruct((_B, _H), _F32),
            jax.ShapeDtypeStruct((_B, 1), jnp.int32),
        ],
        scratch_shapes=[pltpu.VMEM((_B, 1), _F32)],
    )(seq_t, wih_r, wih_z, wih_n, whh_r, whh_z, whh_n,
      br, bz, b_in, b_hn)

    w1 = hand_fc1_w.T.astype(_BF16)
    w2 = hand_fc2_w.T.astype(_BF16)
    wdp = dp_fc1_w.T.astype(_BF16)
    y = pl.pallas_call(
        _head_body,
        out_shape=jax.ShapeDtypeStruct((_B, _NA), _F32),
    )(cards.reshape(_B, _HAND).astype(_BF16), h_last, mask,
      w1, hand_fc1_b.reshape(1, _H), w2, hand_fc2_b.reshape(1, _NA),
      wdp, dp_fc1_b.reshape(1, _NA))
    return y
